# Initial kernel scaffold; baseline (speedup 1.0000x reference)
import jax
import jax.numpy as jnp
from jax.experimental import pallas as pl


def kernel(user_embeds, item_embeds, adj_indices, adj_values):
    raise NotImplementedError("write your pallas kernel here")


# speedup vs baseline: 6.2891x; 6.2891x over previous
import functools

import jax
import jax.numpy as jnp
from jax import lax
from jax.experimental import pallas as pl
from jax.experimental.pallas import tpu as pltpu
from jax.experimental.pallas import tpu_sc as plsc

USER_N = 25000
ITEM_N = 25000
N_NODES = USER_N + ITEM_N
DIM = 64
HALF = DIM // 2
LAYERS = 3
N_EDGES = 800000

NC = 2
NS = 16
SUB = 128
SUBS = 4
CHUNK = SUB * SUBS
NIT = -(-N_EDGES // (NS * CHUNK))
E_PAD = NS * NIT * CHUNK
N_PAD = 50048
ROWS_PER_TILE = N_PAD // NS
ZFULL = ROWS_PER_TILE // CHUNK
ZREM = ROWS_PER_TILE - ZFULL * CHUNK


def _sc_body(x0s, col2, row2, wflat, louts, acc, colb, rowb, wb, gath, sem):
    c = lax.axis_index("c")
    s = lax.axis_index("s")

    zero16 = jnp.zeros((16,), jnp.float32)

    def zb_body(i, _):
        gath[i, pl.ds(0, 16)] = zero16
        gath[i, pl.ds(16, 16)] = zero16
        return 0

    for l in range(LAYERS):
        lax.fori_loop(0, CHUNK, zb_body, 0)
        base = s * ROWS_PER_TILE
        for k in range(ZFULL):
            pltpu.sync_copy(gath, acc.at[pl.ds(base + k * CHUNK, CHUNK)])
        if ZREM:
            pltpu.sync_copy(gath.at[pl.ds(0, ZREM)],
                            acc.at[pl.ds(base + ZFULL * CHUNK, ZREM)])
        plsc.subcore_barrier()

        src = x0s.at[c] if l == 0 else louts.at[l - 1, c]

        def it_body(i, _):
            r0 = s * (NIT * SUBS) + i * SUBS
            pltpu.sync_copy(col2.at[pl.ds(r0, SUBS)], colb)
            pltpu.sync_copy(row2.at[pl.ds(r0, SUBS)], rowb)
            pltpu.sync_copy(wflat.at[pl.ds(r0 * SUB, CHUNK)], wb)
            descs = [
                pltpu.async_copy(src.at[colb.at[j]],
                                 gath.at[pl.ds(j * SUB, SUB)], sem)
                for j in range(SUBS)
            ]
            for d in descs:
                d.wait()

            def g_body(g, _):
                wv = wb[pl.ds(g * 16, 16)]
                dnums = lax.GatherDimensionNumbers(
                    offset_dims=(), collapsed_slice_dims=(0,),
                    start_index_map=(0,))
                for e in range(16):
                    b = lax.gather(
                        wv, jnp.full((16, 1), e, jnp.int32), dnums,
                        slice_sizes=(1,),
                        mode=lax.GatherScatterMode.PROMISE_IN_BOUNDS)
                    r = g * 16 + e
                    gath[r, pl.ds(0, 16)] = gath[r, pl.ds(0, 16)] * b
                    gath[r, pl.ds(16, 16)] = gath[r, pl.ds(16, 16)] * b
                return 0

            lax.fori_loop(0, CHUNK // 16, g_body, 0)

            for j in range(SUBS):
                pltpu.sync_copy(gath.at[pl.ds(j * SUB, SUB)],
                                acc.at[rowb.at[j]], add=True)
            return 0

        lax.fori_loop(0, NIT, it_body, 0)
        plsc.subcore_barrier()

        pltpu.sync_copy(acc.at[pl.ds(s * ROWS_PER_TILE, ROWS_PER_TILE)],
                        louts.at[l, c, pl.ds(s * ROWS_PER_TILE,
                                             ROWS_PER_TILE)])
        plsc.subcore_barrier()


_sc_spmm = pl.kernel(
    _sc_body,
    out_type=jax.ShapeDtypeStruct((LAYERS, NC, N_PAD, HALF), jnp.float32),
    mesh=plsc.VectorSubcoreMesh(core_axis_name="c", subcore_axis_name="s"),
    compiler_params=pltpu.CompilerParams(use_tc_tiling_on_sc=False),
    scratch_types=[
        pltpu.VMEM_SHARED((N_PAD, HALF), jnp.float32),
        pltpu.VMEM((SUBS, SUB), jnp.int32),
        pltpu.VMEM((SUBS, SUB), jnp.int32),
        pltpu.VMEM((CHUNK,), jnp.float32),
        pltpu.VMEM((CHUNK, HALF), jnp.float32),
        pltpu.SemaphoreType.DMA,
    ],
)


def _pool_body(x0_ref, l_ref, o_ref):
    o_ref[...] = x0_ref[...] + l_ref[0] + l_ref[1] + l_ref[2]


_POOL_R = 2000


_pool = pl.pallas_call(
    _pool_body,
    grid=(NC, N_NODES // _POOL_R),
    in_specs=[
        pl.BlockSpec((1, _POOL_R, HALF), lambda c, i: (c, i, 0)),
        pl.BlockSpec((LAYERS, 1, _POOL_R, HALF), lambda c, i: (0, c, i, 0)),
    ],
    out_specs=pl.BlockSpec((1, _POOL_R, HALF), lambda c, i: (c, i, 0)),
    out_shape=jax.ShapeDtypeStruct((NC, N_NODES, HALF), jnp.float32),
)


def kernel(user_embeds, item_embeds, adj_indices, adj_values):
    x0 = jnp.concatenate([user_embeds, item_embeds], axis=0)
    x0s = x0.reshape(N_NODES, NC, HALF).transpose(1, 0, 2)

    row = adj_indices[0].astype(jnp.int32)
    col = adj_indices[1].astype(jnp.int32)
    w = adj_values.astype(jnp.float32)

    pad = E_PAD - N_EDGES
    spread = (jnp.arange(pad, dtype=jnp.int32) % N_NODES)
    col_p = jnp.concatenate([col, spread])
    row_p = jnp.concatenate([row, spread])
    w_p = jnp.concatenate([w, jnp.zeros((pad,), jnp.float32)])

    col2 = col_p.reshape(E_PAD // SUB, SUB)
    row2 = row_p.reshape(E_PAD // SUB, SUB)

    louts = _sc_spmm(x0s, col2, row2, w_p)

    pooled = _pool(x0s, louts)

    out = pooled.transpose(1, 0, 2)
    out = out.reshape(N_NODES, DIM)
    return out[:USER_N], out[USER_N:]


# speedup vs baseline: 9.8836x; 1.5715x over previous
import functools

import jax
import jax.numpy as jnp
from jax import lax
from jax.experimental import pallas as pl
from jax.experimental.pallas import tpu as pltpu
from jax.experimental.pallas import tpu_sc as plsc

USER_N = 25000
ITEM_N = 25000
N_NODES = USER_N + ITEM_N
DIM = 64
HALF = DIM // 2
LAYERS = 3
N_EDGES = 800000

NC = 2
NS = 16
SUB = 128
SUBS = 3
CHUNK = SUB * SUBS
NIT = 132
E_PAD = NS * NIT * CHUNK
IDX_ROWS = E_PAD // SUB + 3 * SUBS
E_ALLOC = IDX_ROWS * SUB
N_PAD = 50048
ROWS_PER_TILE = N_PAD // NS
ZFULL = ROWS_PER_TILE // CHUNK
ZREM = ROWS_PER_TILE - ZFULL * CHUNK


def _sc_body(x0s, col2, row2, wflat, louts, acc, colb, rowb, wb, gath,
             semi0, semi1, semi2, semi3, semg0, semg1, sems0, sems1):
    c = lax.axis_index("c")
    s = lax.axis_index("s")
    semi = (semi0, semi1, semi2, semi3)
    semg = (semg0, semg1)
    sems = (sems0, sems1)

    zero16 = jnp.zeros((16,), jnp.float32)

    def zb_body(i, _):
        gath[0, i, pl.ds(0, 16)] = zero16
        gath[0, i, pl.ds(16, 16)] = zero16
        return 0

    def idx_descs(i, q):
        r0 = s * (NIT * SUBS) + i * SUBS
        return (
            pltpu.make_async_copy(col2.at[pl.ds(r0, SUBS)], colb.at[q],
                                  semi[q]),
            pltpu.make_async_copy(row2.at[pl.ds(r0, SUBS)], rowb.at[q],
                                  semi[q]),
            pltpu.make_async_copy(wflat.at[pl.ds(r0 * SUB, CHUNK)], wb.at[q],
                                  semi[q]),
        )

    def gather_descs(q, b, src):
        return tuple(
            pltpu.make_async_copy(src.at[colb.at[q, j]],
                                  gath.at[b, pl.ds(j * SUB, SUB)], semg[b])
            for j in range(SUBS))

    def scat_start(q, b):
        for j in range(SUBS):
            pltpu.async_copy(gath.at[b, pl.ds(j * SUB, SUB)],
                             acc.at[rowb.at[q, j]], sems[b], add=True)

    def scat_descs(q, b):
        return tuple(
            pltpu.make_async_copy(gath.at[b, pl.ds(j * SUB, SUB)],
                                  acc.at[rowb.at[q, j]], sems[b])
            for j in range(SUBS))

    dnums = lax.GatherDimensionNumbers(
        offset_dims=(), collapsed_slice_dims=(0,), start_index_map=(0,))

    def scale(q, b):
        def g_body(g, _):
            wv = wb[q, pl.ds(g * 16, 16)]
            for e in range(16):
                bc = lax.gather(
                    wv, jnp.full((16, 1), e, jnp.int32), dnums,
                    slice_sizes=(1,),
                    mode=lax.GatherScatterMode.PROMISE_IN_BOUNDS)
                r = g * 16 + e
                gath[b, r, pl.ds(0, 16)] = gath[b, r, pl.ds(0, 16)] * bc
                gath[b, r, pl.ds(16, 16)] = gath[b, r, pl.ds(16, 16)] * bc
            return 0

        lax.fori_loop(0, CHUNK // 16, g_body, 0)

    def half(i, q, src, first):
        b = q % 2
        o = 1 - b
        for d in gather_descs(q, b, src):
            d.wait()
        scale(q, b)
        scat_start(q, b)
        if not first:
            for d in scat_descs((q - 1) % 4, o):
                d.wait()
        for d in idx_descs(i + 3, (q + 3) % 4):
            d.start()
        for d in idx_descs(i + 1, (q + 1) % 4):
            d.wait()
        for d in gather_descs((q + 1) % 4, o, src):
            d.start()

    for l in range(LAYERS):
        lax.fori_loop(0, CHUNK, zb_body, 0)
        base = s * ROWS_PER_TILE
        gz = gath.at[0]
        for k in range(ZFULL):
            pltpu.sync_copy(gz, acc.at[pl.ds(base + k * CHUNK, CHUNK)])
        if ZREM:
            pltpu.sync_copy(gz.at[pl.ds(0, ZREM)],
                            acc.at[pl.ds(base + ZFULL * CHUNK, ZREM)])
        plsc.subcore_barrier()

        src = x0s.at[c] if l == 0 else louts.at[l - 1, c]

        for q0 in range(3):
            for d in idx_descs(q0, q0):
                d.start()
        for d in idx_descs(0, 0):
            d.wait()
        for d in gather_descs(0, 0, src):
            d.start()
        half(0, 0, src, True)
        half(1, 1, src, False)
        half(2, 2, src, False)
        half(3, 3, src, False)

        def quad(p, _):
            i = 4 + 4 * p
            half(i, 0, src, False)
            half(i + 1, 1, src, False)
            half(i + 2, 2, src, False)
            half(i + 3, 3, src, False)
            return 0

        lax.fori_loop(0, (NIT - 4) // 4, quad, 0)

        for d in scat_descs(3, 1):
            d.wait()
        for d in gather_descs(0, 0, src):
            d.wait()
        for d in idx_descs(NIT + 1, 1):
            d.wait()
        for d in idx_descs(NIT + 2, 2):
            d.wait()
        plsc.subcore_barrier()

        pltpu.sync_copy(acc.at[pl.ds(base, ROWS_PER_TILE)],
                        louts.at[l, c, pl.ds(base, ROWS_PER_TILE)])
        plsc.subcore_barrier()


_sc_spmm = pl.kernel(
    _sc_body,
    out_type=jax.ShapeDtypeStruct((LAYERS, NC, N_PAD, HALF), jnp.float32),
    mesh=plsc.VectorSubcoreMesh(core_axis_name="c", subcore_axis_name="s"),
    compiler_params=pltpu.CompilerParams(use_tc_tiling_on_sc=False),
    scratch_types=[
        pltpu.VMEM_SHARED((N_PAD, HALF), jnp.float32),
        pltpu.VMEM((4, SUBS, SUB), jnp.int32),
        pltpu.VMEM((4, SUBS, SUB), jnp.int32),
        pltpu.VMEM((4, CHUNK), jnp.float32),
        pltpu.VMEM((2, CHUNK, HALF), jnp.float32),
        pltpu.SemaphoreType.DMA,
        pltpu.SemaphoreType.DMA,
        pltpu.SemaphoreType.DMA,
        pltpu.SemaphoreType.DMA,
        pltpu.SemaphoreType.DMA,
        pltpu.SemaphoreType.DMA,
        pltpu.SemaphoreType.DMA,
        pltpu.SemaphoreType.DMA,
    ],
)


def _pool_body(x0_ref, l_ref, o_ref):
    o_ref[...] = x0_ref[...] + l_ref[0] + l_ref[1] + l_ref[2]


_POOL_R = 2000


_pool = pl.pallas_call(
    _pool_body,
    grid=(NC, N_NODES // _POOL_R),
    in_specs=[
        pl.BlockSpec((1, _POOL_R, HALF), lambda c, i: (c, i, 0)),
        pl.BlockSpec((LAYERS, 1, _POOL_R, HALF), lambda c, i: (0, c, i, 0)),
    ],
    out_specs=pl.BlockSpec((1, _POOL_R, HALF), lambda c, i: (c, i, 0)),
    out_shape=jax.ShapeDtypeStruct((NC, N_NODES, HALF), jnp.float32),
)


def kernel(user_embeds, item_embeds, adj_indices, adj_values):
    x0 = jnp.concatenate([user_embeds, item_embeds], axis=0)
    x0s = x0.reshape(N_NODES, NC, HALF).transpose(1, 0, 2)

    row = adj_indices[0].astype(jnp.int32)
    col = adj_indices[1].astype(jnp.int32)
    w = adj_values.astype(jnp.float32)

    pad = E_ALLOC - N_EDGES
    spread = (jnp.arange(pad, dtype=jnp.int32) % N_NODES)
    col_p = jnp.concatenate([col, spread])
    row_p = jnp.concatenate([row, spread])
    w_p = jnp.concatenate([w, jnp.zeros((pad,), jnp.float32)])

    col2 = col_p.reshape(IDX_ROWS, SUB)
    row2 = row_p.reshape(IDX_ROWS, SUB)

    louts = _sc_spmm(x0s, col2, row2, w_p)

    pooled = _pool(x0s, louts)

    out = pooled.transpose(1, 0, 2)
    out = out.reshape(N_NODES, DIM)
    return out[:USER_N], out[USER_N:]


# speedup vs baseline: 12.1086x; 1.2251x over previous
import functools

import jax
import jax.numpy as jnp
from jax import lax
from jax.experimental import pallas as pl
from jax.experimental.pallas import tpu as pltpu
from jax.experimental.pallas import tpu_sc as plsc

USER_N = 25000
ITEM_N = 25000
N_NODES = USER_N + ITEM_N
DIM = 64
HALF = DIM // 2
LAYERS = 3
N_EDGES = 800000

NC = 2
NS = 16
SUB = 128
SUBS = 3
CHUNK = SUB * SUBS
NIT = 132
E_PAD = NS * NIT * CHUNK
IDX_ROWS = E_PAD // SUB + 3 * SUBS
E_ALLOC = IDX_ROWS * SUB
N_PAD = 50048
ROWS_PER_TILE = N_PAD // NS
ZFULL = ROWS_PER_TILE // CHUNK
ZREM = ROWS_PER_TILE - ZFULL * CHUNK


def _sc_body(x0s, col2, row2, wflat, louts, acc, colb, rowb, wb, gath,
             semi0, semi1, semi2, semi3, semg0, semg1, sems0, sems1):
    c = lax.axis_index("c")
    s = lax.axis_index("s")
    semi = (semi0, semi1, semi2, semi3)
    semg = (semg0, semg1)
    sems = (sems0, sems1)

    zero16 = jnp.zeros((16,), jnp.float32)

    def zb_body(i, _):
        gath[0, i, pl.ds(0, 16)] = zero16
        gath[0, i, pl.ds(16, 16)] = zero16
        return 0

    def idx_descs(i, q):
        r0 = s * (NIT * SUBS) + i * SUBS
        return (
            pltpu.make_async_copy(col2.at[pl.ds(r0, SUBS)], colb.at[q],
                                  semi[q]),
            pltpu.make_async_copy(row2.at[pl.ds(r0, SUBS)], rowb.at[q],
                                  semi[q]),
            pltpu.make_async_copy(wflat.at[pl.ds(r0 * SUB, CHUNK)], wb.at[q],
                                  semi[q]),
        )

    def gather_descs(q, b, src):
        return tuple(
            pltpu.make_async_copy(src.at[colb.at[q, j]],
                                  gath.at[b, pl.ds(j * SUB, SUB)], semg[b])
            for j in range(SUBS))

    def scat_start(q, b):
        for j in range(SUBS):
            pltpu.async_copy(gath.at[b, pl.ds(j * SUB, SUB)],
                             acc.at[rowb.at[q, j]], sems[b], add=True)

    def scat_descs(q, b):
        return tuple(
            pltpu.make_async_copy(gath.at[b, pl.ds(j * SUB, SUB)],
                                  acc.at[rowb.at[q, j]], sems[b])
            for j in range(SUBS))

    dnums = lax.GatherDimensionNumbers(
        offset_dims=(), collapsed_slice_dims=(0,), start_index_map=(0,))

    def scale(q, b):
        def g_body(g, _):
            wv = wb[q, pl.ds(g * 16, 16)]
            for e in range(16):
                bc = lax.gather(
                    wv, jnp.full((16, 1), e, jnp.int32), dnums,
                    slice_sizes=(1,),
                    mode=lax.GatherScatterMode.PROMISE_IN_BOUNDS)
                r = g * 16 + e
                gath[b, r, pl.ds(0, 16)] = gath[b, r, pl.ds(0, 16)] * bc
                gath[b, r, pl.ds(16, 16)] = gath[b, r, pl.ds(16, 16)] * bc
            return 0

        lax.fori_loop(0, CHUNK // 16, g_body, 0)

    def half(i, q, src, first):
        b = q % 2
        o = 1 - b
        for d in gather_descs(q, b, src):
            d.wait()
        if not first:
            for d in scat_descs((q - 1) % 4, o):
                d.wait()
        for d in idx_descs(i + 1, (q + 1) % 4):
            d.wait()
        for d in gather_descs((q + 1) % 4, o, src):
            d.start()
        for d in idx_descs(i + 3, (q + 3) % 4):
            d.start()
        scale(q, b)
        scat_start(q, b)

    for l in range(LAYERS):
        lax.fori_loop(0, CHUNK, zb_body, 0)
        base = s * ROWS_PER_TILE
        gz = gath.at[0]
        for k in range(ZFULL):
            pltpu.sync_copy(gz, acc.at[pl.ds(base + k * CHUNK, CHUNK)])
        if ZREM:
            pltpu.sync_copy(gz.at[pl.ds(0, ZREM)],
                            acc.at[pl.ds(base + ZFULL * CHUNK, ZREM)])
        plsc.subcore_barrier()

        src = x0s.at[c] if l == 0 else louts.at[l - 1, c]

        for q0 in range(3):
            for d in idx_descs(q0, q0):
                d.start()
        for d in idx_descs(0, 0):
            d.wait()
        for d in gather_descs(0, 0, src):
            d.start()
        half(0, 0, src, True)
        half(1, 1, src, False)
        half(2, 2, src, False)
        half(3, 3, src, False)

        def quad(p, _):
            i = 4 + 4 * p
            half(i, 0, src, False)
            half(i + 1, 1, src, False)
            half(i + 2, 2, src, False)
            half(i + 3, 3, src, False)
            return 0

        lax.fori_loop(0, (NIT - 4) // 4, quad, 0)

        for d in scat_descs(3, 1):
            d.wait()
        for d in gather_descs(0, 0, src):
            d.wait()
        for d in idx_descs(NIT + 1, 1):
            d.wait()
        for d in idx_descs(NIT + 2, 2):
            d.wait()
        plsc.subcore_barrier()

        pltpu.sync_copy(acc.at[pl.ds(base, ROWS_PER_TILE)],
                        louts.at[l, c, pl.ds(base, ROWS_PER_TILE)])
        plsc.subcore_barrier()


_sc_spmm = pl.kernel(
    _sc_body,
    out_type=jax.ShapeDtypeStruct((LAYERS, NC, N_PAD, HALF), jnp.float32),
    mesh=plsc.VectorSubcoreMesh(core_axis_name="c", subcore_axis_name="s"),
    compiler_params=pltpu.CompilerParams(use_tc_tiling_on_sc=False),
    scratch_types=[
        pltpu.VMEM_SHARED((N_PAD, HALF), jnp.float32),
        pltpu.VMEM((4, SUBS, SUB), jnp.int32),
        pltpu.VMEM((4, SUBS, SUB), jnp.int32),
        pltpu.VMEM((4, CHUNK), jnp.float32),
        pltpu.VMEM((2, CHUNK, HALF), jnp.float32),
        pltpu.SemaphoreType.DMA,
        pltpu.SemaphoreType.DMA,
        pltpu.SemaphoreType.DMA,
        pltpu.SemaphoreType.DMA,
        pltpu.SemaphoreType.DMA,
        pltpu.SemaphoreType.DMA,
        pltpu.SemaphoreType.DMA,
        pltpu.SemaphoreType.DMA,
    ],
)


def _pool_body(x0_ref, l_ref, o_ref):
    o_ref[...] = x0_ref[...] + l_ref[0] + l_ref[1] + l_ref[2]


_POOL_R = 2000


_pool = pl.pallas_call(
    _pool_body,
    grid=(NC, N_NODES // _POOL_R),
    in_specs=[
        pl.BlockSpec((1, _POOL_R, HALF), lambda c, i: (c, i, 0)),
        pl.BlockSpec((LAYERS, 1, _POOL_R, HALF), lambda c, i: (0, c, i, 0)),
    ],
    out_specs=pl.BlockSpec((1, _POOL_R, HALF), lambda c, i: (c, i, 0)),
    out_shape=jax.ShapeDtypeStruct((NC, N_NODES, HALF), jnp.float32),
)


def kernel(user_embeds, item_embeds, adj_indices, adj_values):
    x0 = jnp.concatenate([user_embeds, item_embeds], axis=0)
    x0s = x0.reshape(N_NODES, NC, HALF).transpose(1, 0, 2)

    row = adj_indices[0].astype(jnp.int32)
    col = adj_indices[1].astype(jnp.int32)
    w = adj_values.astype(jnp.float32)

    pad = E_ALLOC - N_EDGES
    spread = (jnp.arange(pad, dtype=jnp.int32) % N_NODES)
    col_p = jnp.concatenate([col, spread])
    row_p = jnp.concatenate([row, spread])
    w_p = jnp.concatenate([w, jnp.zeros((pad,), jnp.float32)])

    col2 = col_p.reshape(IDX_ROWS, SUB)
    row2 = row_p.reshape(IDX_ROWS, SUB)

    louts = _sc_spmm(x0s, col2, row2, w_p)

    pooled = _pool(x0s, louts)

    out = pooled.transpose(1, 0, 2)
    out = out.reshape(N_NODES, DIM)
    return out[:USER_N], out[USER_N:]


# speedup vs baseline: 12.1800x; 1.0059x over previous
import functools

import jax
import jax.numpy as jnp
from jax import lax
from jax.experimental import pallas as pl
from jax.experimental.pallas import tpu as pltpu
from jax.experimental.pallas import tpu_sc as plsc

USER_N = 25000
ITEM_N = 25000
N_NODES = USER_N + ITEM_N
DIM = 64
HALF = DIM // 2
LAYERS = 3
N_EDGES = 800000

NC = 2
NS = 16
CHUNK = 384
NIT = 132
E_PAD = NS * NIT * CHUNK
N_CHUNKS = E_PAD // CHUNK + 3
E_ALLOC = N_CHUNKS * CHUNK
N_PAD = 50048
ROWS_PER_TILE = N_PAD // NS
ZFULL = ROWS_PER_TILE // CHUNK
ZREM = ROWS_PER_TILE - ZFULL * CHUNK


def _sc_body(x0s, col3, row3, wflat, louts, acc, colb, rowb, wb, gath,
             semi0, semi1, semi2, semi3, semg0, semg1, sems0, sems1):
    c = lax.axis_index("c")
    s = lax.axis_index("s")
    semi = (semi0, semi1, semi2, semi3)
    semg = (semg0, semg1)
    sems = (sems0, sems1)

    zero16 = jnp.zeros((16,), jnp.float32)

    def zb_body(i, _):
        gath[0, i, pl.ds(0, 16)] = zero16
        gath[0, i, pl.ds(16, 16)] = zero16
        return 0

    def idx_descs(i, q):
        ch = s * NIT + i
        return (
            pltpu.make_async_copy(col3.at[ch], colb.at[q], semi[q]),
            pltpu.make_async_copy(row3.at[ch], rowb.at[q], semi[q]),
            pltpu.make_async_copy(wflat.at[pl.ds(ch * CHUNK, CHUNK)],
                                  wb.at[q], semi[q]),
        )

    def gather_descs(q, b, src):
        return (
            pltpu.make_async_copy(src.at[colb.at[q]], gath.at[b], semg[b]),)

    def scat_start(q, b):
        pltpu.async_copy(gath.at[b], acc.at[rowb.at[q]], sems[b], add=True)

    def scat_descs(q, b):
        return (
            pltpu.make_async_copy(gath.at[b], acc.at[rowb.at[q]], sems[b]),)

    dnums = lax.GatherDimensionNumbers(
        offset_dims=(), collapsed_slice_dims=(0,), start_index_map=(0,))

    def scale(q, b):
        def g_body(g, _):
            wv = wb[q, pl.ds(g * 16, 16)]
            for e in range(16):
                bc = lax.gather(
                    wv, jnp.full((16, 1), e, jnp.int32), dnums,
                    slice_sizes=(1,),
                    mode=lax.GatherScatterMode.PROMISE_IN_BOUNDS)
                r = g * 16 + e
                gath[b, r, pl.ds(0, 16)] = gath[b, r, pl.ds(0, 16)] * bc
                gath[b, r, pl.ds(16, 16)] = gath[b, r, pl.ds(16, 16)] * bc
            return 0

        lax.fori_loop(0, CHUNK // 16, g_body, 0)

    def half(i, q, src, first):
        b = q % 2
        o = 1 - b
        for d in gather_descs(q, b, src):
            d.wait()
        if not first:
            for d in scat_descs((q - 1) % 4, o):
                d.wait()
        for d in idx_descs(i + 1, (q + 1) % 4):
            d.wait()
        for d in gather_descs((q + 1) % 4, o, src):
            d.start()
        for d in idx_descs(i + 3, (q + 3) % 4):
            d.start()
        scale(q, b)
        scat_start(q, b)

    for l in range(LAYERS):
        lax.fori_loop(0, CHUNK, zb_body, 0)
        base = s * ROWS_PER_TILE
        gz = gath.at[0]
        zd = []
        for k in range(ZFULL):
            zd.append(pltpu.make_async_copy(
                gz, acc.at[pl.ds(base + k * CHUNK, CHUNK)], semg0))
        if ZREM:
            zd.append(pltpu.make_async_copy(
                gz.at[pl.ds(0, ZREM)],
                acc.at[pl.ds(base + ZFULL * CHUNK, ZREM)], semg0))
        for d in zd:
            d.start()
        for d in zd:
            d.wait()
        plsc.subcore_barrier()

        src = x0s.at[c] if l == 0 else louts.at[l - 1, c]

        for q0 in range(3):
            for d in idx_descs(q0, q0):
                d.start()
        for d in idx_descs(0, 0):
            d.wait()
        for d in gather_descs(0, 0, src):
            d.start()
        half(0, 0, src, True)
        half(1, 1, src, False)
        half(2, 2, src, False)
        half(3, 3, src, False)

        def quad(p, _):
            i = 4 + 4 * p
            half(i, 0, src, False)
            half(i + 1, 1, src, False)
            half(i + 2, 2, src, False)
            half(i + 3, 3, src, False)
            return 0

        lax.fori_loop(0, (NIT - 4) // 4, quad, 0)

        for d in scat_descs(3, 1):
            d.wait()
        for d in gather_descs(0, 0, src):
            d.wait()
        for d in idx_descs(NIT + 1, 1):
            d.wait()
        for d in idx_descs(NIT + 2, 2):
            d.wait()
        plsc.subcore_barrier()

        pltpu.sync_copy(acc.at[pl.ds(base, ROWS_PER_TILE)],
                        louts.at[l, c, pl.ds(base, ROWS_PER_TILE)])
        plsc.subcore_barrier()


_sc_spmm = pl.kernel(
    _sc_body,
    out_type=jax.ShapeDtypeStruct((LAYERS, NC, N_PAD, HALF), jnp.float32),
    mesh=plsc.VectorSubcoreMesh(core_axis_name="c", subcore_axis_name="s"),
    compiler_params=pltpu.CompilerParams(use_tc_tiling_on_sc=False),
    scratch_types=[
        pltpu.VMEM_SHARED((N_PAD, HALF), jnp.float32),
        pltpu.VMEM((4, CHUNK), jnp.int32),
        pltpu.VMEM((4, CHUNK), jnp.int32),
        pltpu.VMEM((4, CHUNK), jnp.float32),
        pltpu.VMEM((2, CHUNK, HALF), jnp.float32),
        pltpu.SemaphoreType.DMA,
        pltpu.SemaphoreType.DMA,
        pltpu.SemaphoreType.DMA,
        pltpu.SemaphoreType.DMA,
        pltpu.SemaphoreType.DMA,
        pltpu.SemaphoreType.DMA,
        pltpu.SemaphoreType.DMA,
        pltpu.SemaphoreType.DMA,
    ],
)


def _pool_body(x0_ref, l_ref, o_ref):
    o_ref[...] = x0_ref[...] + l_ref[0] + l_ref[1] + l_ref[2]


_POOL_R = 2000


_pool = pl.pallas_call(
    _pool_body,
    grid=(NC, N_NODES // _POOL_R),
    in_specs=[
        pl.BlockSpec((1, _POOL_R, HALF), lambda c, i: (c, i, 0)),
        pl.BlockSpec((LAYERS, 1, _POOL_R, HALF), lambda c, i: (0, c, i, 0)),
    ],
    out_specs=pl.BlockSpec((1, _POOL_R, HALF), lambda c, i: (c, i, 0)),
    out_shape=jax.ShapeDtypeStruct((NC, N_NODES, HALF), jnp.float32),
)


def kernel(user_embeds, item_embeds, adj_indices, adj_values):
    x0 = jnp.concatenate([user_embeds, item_embeds], axis=0)
    x0s = x0.reshape(N_NODES, NC, HALF).transpose(1, 0, 2)

    row = adj_indices[0].astype(jnp.int32)
    col = adj_indices[1].astype(jnp.int32)
    w = adj_values.astype(jnp.float32)

    pad = E_ALLOC - N_EDGES
    spread = (jnp.arange(pad, dtype=jnp.int32) % N_NODES)
    col_p = jnp.concatenate([col, spread])
    row_p = jnp.concatenate([row, spread])
    w_p = jnp.concatenate([w, jnp.zeros((pad,), jnp.float32)])

    col3 = col_p.reshape(N_CHUNKS, CHUNK)
    row3 = row_p.reshape(N_CHUNKS, CHUNK)

    louts = _sc_spmm(x0s, col3, row3, w_p)

    pooled = _pool(x0s, louts)

    out = pooled.transpose(1, 0, 2)
    out = out.reshape(N_NODES, DIM)
    return out[:USER_N], out[USER_N:]
